# Initial kernel scaffold; baseline (speedup 1.0000x reference)
import jax
import jax.numpy as jnp
from jax.experimental import pallas as pl


def kernel(x, pos, batch, W_lin, W_src, W_dst, W_pos, b_pos, W_out, b_out):
    raise NotImplementedError("write your pallas kernel here")


# speedup vs baseline: 6.1068x; 6.1068x over previous
import functools

import jax
import jax.numpy as jnp
from jax import lax
from jax.experimental import pallas as pl
from jax.experimental.pallas import tpu as pltpu, tpu_sc as plsc

N = 10000
D = 128
K = 16
NPAD = 10240
CPAD = 10112
RB = 200
PB = 1000

_f32 = jnp.float32


def _pre_body(x_ref, pos_ref, wl_ref, ws_ref, wd_ref, wp_ref, bp_ref,
              a_ref, b_ref, c_ref, v_ref):
    xb = x_ref[...]
    pb = pos_ref[...]
    pn = jnp.dot(pb, wp_ref[...], preferred_element_type=_f32)
    q = jnp.dot(xb, wd_ref[...], preferred_element_type=_f32)
    kk = jnp.dot(xb, ws_ref[...], preferred_element_type=_f32)
    v = jnp.dot(xb, wl_ref[...], preferred_element_type=_f32)
    bp = bp_ref[...]
    a_ref[...] = q + pn + bp
    b_ref[...] = kk + pn
    c_ref[...] = pn + bp
    v_ref[...] = v - pn


def _precompute(x, pos_p, W_lin, W_src, W_dst, W_pos_p, b_pos2):
    nblk = N // PB
    row = pl.BlockSpec((PB, D), lambda i: (i, 0))
    full = pl.BlockSpec((D, D), lambda i: (0, 0))
    bias = pl.BlockSpec((1, D), lambda i: (0, 0))
    out = jax.ShapeDtypeStruct((N, D), _f32)
    return pl.pallas_call(
        _pre_body,
        grid=(nblk,),
        in_specs=[row, row, full, full, full, full, bias],
        out_specs=[row, row, row, row],
        out_shape=[out, out, out, out],
    )(x, pos_p, W_lin, W_src, W_dst, W_pos_p, b_pos2)


def _knn_body(pos_ref, post_ref, idx_ref):
    pb = pos_ref[...]
    pt = post_ref[...]
    dot = jnp.dot(pb, pt, preferred_element_type=_f32)
    sqb = jnp.sum(pb * pb, axis=1, keepdims=True)
    sqt = jnp.sum(pt * pt, axis=0, keepdims=True)
    d2 = sqb + sqt - 2.0 * dot
    col = lax.broadcasted_iota(jnp.int32, d2.shape, 1)
    d2 = jnp.where(col < N, d2, jnp.inf)
    ki = lax.broadcasted_iota(jnp.int32, (RB, K), 1)
    idxb = jnp.zeros((RB, K), jnp.int32)
    for k in range(K):
        m = jnp.min(d2, axis=1, keepdims=True)
        am = jnp.min(jnp.where(d2 == m, col, jnp.int32(2 ** 30)),
                     axis=1, keepdims=True)
        idxb = jnp.where(ki == k, am, idxb)
        d2 = jnp.where(col == am, jnp.inf, d2)
    idx_ref[...] = idxb


def _knn(pos_p, post_p):
    nblk = N // RB
    return pl.pallas_call(
        _knn_body,
        grid=(nblk,),
        in_specs=[pl.BlockSpec((RB, D), lambda i: (i, 0)),
                  pl.BlockSpec((D, CPAD), lambda i: (0, 0))],
        out_specs=pl.BlockSpec((RB, K), lambda i: (i, 0)),
        out_shape=jax.ShapeDtypeStruct((N, K), jnp.int32),
    )(pos_p, post_p)


NB = 8
ROWS_PER_W = NPAD // 32
NBATCH = ROWS_PER_W // NB


def _gather_softmax(a_p, b_p, c_p, v_p, idx_flat):
    mesh = plsc.VectorSubcoreMesh(core_axis_name="c", subcore_axis_name="s",
                                  num_cores=2, num_subcores=16)

    @functools.partial(
        pl.kernel,
        mesh=mesh,
        out_type=jax.ShapeDtypeStruct((NPAD, D), _f32),
        scratch_types=[
            pltpu.VMEM((NB * K,), jnp.int32),
            pltpu.VMEM((NB * K, D), _f32),
            pltpu.VMEM((NB * K, D), _f32),
            pltpu.VMEM((NB, D), _f32),
            pltpu.VMEM((NB, D), _f32),
            pltpu.VMEM((NB, D), _f32),
            pltpu.SemaphoreType.DMA,
        ],
    )
    def run(a_h, b_h, c_h, v_h, idx_h, out_h,
            idx_v, brow_v, vrow_v, a_v, c_v, rep_v, sem):
        cid = lax.axis_index("c")
        sid = lax.axis_index("s")
        wid = sid * 2 + cid
        row0 = wid * ROWS_PER_W

        def batch_body(t, carry):
            r0 = row0 + t * NB
            pltpu.sync_copy(idx_h.at[pl.ds(r0 * K, NB * K)], idx_v)
            pltpu.async_copy(b_h.at[idx_v], brow_v, sem).wait()
            pltpu.async_copy(v_h.at[idx_v], vrow_v, sem).wait()
            pltpu.sync_copy(a_h.at[pl.ds(r0, NB)], a_v)
            pltpu.sync_copy(c_h.at[pl.ds(r0, NB)], c_v)

            def row_body(i, carry2):
                for ch in range(D // 16):
                    sl = pl.ds(ch * 16, 16)
                    a = a_v[i, sl]
                    m = a - brow_v[i * K, sl]
                    for k in range(1, K):
                        m = jnp.maximum(m, a - brow_v[i * K + k, sl])
                    ssum = jnp.zeros((16,), _f32)
                    acc = jnp.zeros((16,), _f32)
                    for k in range(K):
                        e = jnp.exp(a - brow_v[i * K + k, sl] - m)
                        ssum = ssum + e
                        acc = acc + e * vrow_v[i * K + k, sl]
                    rep_v[i, sl] = (c_v[i, sl] * ssum + acc) / (ssum + 1e-16)
                return carry2

            lax.fori_loop(0, NB, row_body, 0)
            pltpu.sync_copy(rep_v, out_h.at[pl.ds(r0, NB)])
            return carry

        lax.fori_loop(0, NBATCH, batch_body, 0)

    return run(a_p, b_p, c_p, v_p, idx_flat)


def _out_body(rep_ref, wo_ref, bo_ref, o_ref):
    o_ref[...] = (jnp.dot(rep_ref[...], wo_ref[...],
                          preferred_element_type=_f32) + bo_ref[...])


def _out_mm(rep, W_out, b_out2):
    nblk = N // PB
    return pl.pallas_call(
        _out_body,
        grid=(nblk,),
        in_specs=[pl.BlockSpec((PB, D), lambda i: (i, 0)),
                  pl.BlockSpec((D, D), lambda i: (0, 0)),
                  pl.BlockSpec((1, D), lambda i: (0, 0))],
        out_specs=pl.BlockSpec((PB, D), lambda i: (i, 0)),
        out_shape=jax.ShapeDtypeStruct((N, D), _f32),
    )(rep, W_out, b_out2)


def kernel(x, pos, batch, W_lin, W_src, W_dst, W_pos, b_pos, W_out, b_out):
    del batch
    pos_p = jnp.pad(pos, ((0, 0), (0, D - 3)))
    post_p = jnp.pad(pos.T, ((0, D - 3), (0, CPAD - N)))
    W_pos_p = jnp.pad(W_pos, ((0, D - 3), (0, 0)))
    b_pos2 = b_pos[None, :]
    b_out2 = b_out[None, :]

    a, b, c, v = _precompute(x, pos_p, W_lin, W_src, W_dst, W_pos_p, b_pos2)
    idx = _knn(pos_p, post_p)

    pad = ((0, NPAD - N), (0, 0))
    a_p = jnp.pad(a, pad)
    b_p = jnp.pad(b, pad)
    c_p = jnp.pad(c, pad)
    v_p = jnp.pad(v, pad)
    idx_flat = jnp.pad(idx, pad).reshape(-1)

    rep = _gather_softmax(a_p, b_p, c_p, v_p, idx_flat)[:N]
    return _out_mm(rep, W_out, b_out2)


# speedup vs baseline: 6.7401x; 1.1037x over previous
import functools

import jax
import jax.numpy as jnp
from jax import lax
from jax.experimental import pallas as pl
from jax.experimental.pallas import tpu as pltpu, tpu_sc as plsc

N = 10000
D = 128
K = 16
NPAD = 10240
CPAD = 10112
RB = 200
PB = 1000

_f32 = jnp.float32


def _pre_body(x_ref, pos_ref, wl_ref, ws_ref, wd_ref, wp_ref, bp_ref,
              a_ref, b_ref, c_ref, v_ref):
    xb = x_ref[...]
    pb = pos_ref[...]
    pn = jnp.dot(pb, wp_ref[...], preferred_element_type=_f32)
    q = jnp.dot(xb, wd_ref[...], preferred_element_type=_f32)
    kk = jnp.dot(xb, ws_ref[...], preferred_element_type=_f32)
    v = jnp.dot(xb, wl_ref[...], preferred_element_type=_f32)
    bp = bp_ref[...]
    a_ref[...] = q + pn + bp
    b_ref[...] = kk + pn
    c_ref[...] = pn + bp
    v_ref[...] = v - pn


def _precompute(x, pos_p, W_lin, W_src, W_dst, W_pos_p, b_pos2):
    nblk = N // PB
    row = pl.BlockSpec((PB, D), lambda i: (i, 0))
    full = pl.BlockSpec((D, D), lambda i: (0, 0))
    bias = pl.BlockSpec((1, D), lambda i: (0, 0))
    out = jax.ShapeDtypeStruct((N, D), _f32)
    return pl.pallas_call(
        _pre_body,
        grid=(nblk,),
        in_specs=[row, row, full, full, full, full, bias],
        out_specs=[row, row, row, row],
        out_shape=[out, out, out, out],
    )(x, pos_p, W_lin, W_src, W_dst, W_pos_p, b_pos2)


def _knn_body(pos_ref, post_ref, idx_ref):
    pb = pos_ref[...]
    pt = post_ref[...]
    dot = jnp.dot(pb, pt, preferred_element_type=_f32)
    sqb = jnp.sum(pb * pb, axis=1, keepdims=True)
    sqt = jnp.sum(pt * pt, axis=0, keepdims=True)
    d2 = sqb + sqt - 2.0 * dot
    col = lax.broadcasted_iota(jnp.int32, d2.shape, 1)
    d2 = jnp.where(col < N, d2, jnp.inf)
    ki = lax.broadcasted_iota(jnp.int32, (RB, K), 1)
    idxb = jnp.zeros((RB, K), jnp.int32)
    for k in range(K):
        m = jnp.min(d2, axis=1, keepdims=True)
        am = jnp.min(jnp.where(d2 == m, col, jnp.int32(2 ** 30)),
                     axis=1, keepdims=True)
        idxb = jnp.where(ki == k, am, idxb)
        d2 = jnp.where(col == am, jnp.inf, d2)
    idx_ref[...] = idxb


def _knn(pos_p, post_p):
    nblk = N // RB
    return pl.pallas_call(
        _knn_body,
        grid=(nblk,),
        in_specs=[pl.BlockSpec((RB, D), lambda i: (i, 0)),
                  pl.BlockSpec((D, CPAD), lambda i: (0, 0))],
        out_specs=pl.BlockSpec((RB, K), lambda i: (i, 0)),
        out_shape=jax.ShapeDtypeStruct((N, K), jnp.int32),
    )(pos_p, post_p)


NB = 8
ROWS_PER_W = NPAD // 32
NBATCH = ROWS_PER_W // NB


NBK = NB * K


def _gather_softmax(a_p, b_p, c_p, v_p, idx_flat):
    mesh = plsc.VectorSubcoreMesh(core_axis_name="c", subcore_axis_name="s",
                                  num_cores=2, num_subcores=16)

    @functools.partial(
        pl.kernel,
        mesh=mesh,
        out_type=jax.ShapeDtypeStruct((NPAD, D), _f32),
        scratch_types=[
            pltpu.VMEM((ROWS_PER_W * K,), jnp.int32),
            pltpu.VMEM((2, NBK, D), _f32),
            pltpu.VMEM((2, NBK, D), _f32),
            pltpu.VMEM((2, NB, D), _f32),
            pltpu.VMEM((2, NB, D), _f32),
            pltpu.VMEM((2, NB, D), _f32),
            pltpu.SemaphoreType.DMA,
            pltpu.SemaphoreType.DMA,
            pltpu.SemaphoreType.DMA,
            pltpu.SemaphoreType.DMA,
        ],
    )
    def run(a_h, b_h, c_h, v_h, idx_h, out_h,
            idx_all, brow, vrow, av, cv, repv, si0, si1, so0, so1):
        cid = lax.axis_index("c")
        sid = lax.axis_index("s")
        wid = sid * 2 + cid
        row0 = wid * ROWS_PER_W
        sem_in = [si0, si1]
        sem_o = [so0, so1]
        pltpu.sync_copy(idx_h.at[pl.ds(row0 * K, ROWS_PER_W * K)], idx_all)

        def issue(t, buf):
            r0 = row0 + t * NB
            isl = idx_all.at[pl.ds(t * NBK, NBK)]
            pltpu.async_copy(b_h.at[isl], brow.at[buf], sem_in[buf])
            pltpu.async_copy(v_h.at[isl], vrow.at[buf], sem_in[buf])
            pltpu.async_copy(a_h.at[pl.ds(r0, NB)], av.at[buf], sem_in[buf])
            pltpu.async_copy(c_h.at[pl.ds(r0, NB)], cv.at[buf], sem_in[buf])

        def drain_in(t, buf):
            r0 = row0 + t * NB
            isl = idx_all.at[pl.ds(t * NBK, NBK)]
            pltpu.make_async_copy(b_h.at[isl], brow.at[buf], sem_in[buf]).wait()
            pltpu.make_async_copy(v_h.at[isl], vrow.at[buf], sem_in[buf]).wait()
            pltpu.make_async_copy(a_h.at[pl.ds(r0, NB)], av.at[buf], sem_in[buf]).wait()
            pltpu.make_async_copy(c_h.at[pl.ds(r0, NB)], cv.at[buf], sem_in[buf]).wait()

        def compute(buf):
            def row_body(i, c2):
                for ch in range(D // 16):
                    sl = pl.ds(ch * 16, 16)
                    a = av[buf, i, sl]
                    alphas = [a - brow[buf, i * K + k, sl] for k in range(K)]
                    m = alphas[0]
                    for k in range(1, K):
                        m = jnp.maximum(m, alphas[k])
                    ssum = jnp.zeros((16,), _f32)
                    acc = jnp.zeros((16,), _f32)
                    for k in range(K):
                        e = jnp.exp(alphas[k] - m)
                        ssum = ssum + e
                        acc = acc + e * vrow[buf, i * K + k, sl]
                    repv[buf, i, sl] = (cv[buf, i, sl] * ssum + acc) / (ssum + 1e-16)
                return c2

            lax.fori_loop(0, NB, row_body, 0)

        issue(0, 0)

        def pair_body(p, carry):
            for sub in range(2):
                t = p * 2 + sub
                buf = sub
                drain_in(t, buf)

                @pl.when(t + 1 < NBATCH)
                def _():
                    issue(t + 1, 1 - buf)

                @pl.when(t >= 2)
                def _():
                    r0p = row0 + (t - 2) * NB
                    pltpu.make_async_copy(
                        repv.at[buf], out_h.at[pl.ds(r0p, NB)],
                        sem_o[buf]).wait()

                compute(buf)
                r0 = row0 + t * NB
                pltpu.async_copy(repv.at[buf], out_h.at[pl.ds(r0, NB)],
                                 sem_o[buf])
            return carry

        lax.fori_loop(0, NBATCH // 2, pair_body, 0)
        pltpu.make_async_copy(
            repv.at[0], out_h.at[pl.ds(row0 + (NBATCH - 2) * NB, NB)],
            sem_o[0]).wait()
        pltpu.make_async_copy(
            repv.at[1], out_h.at[pl.ds(row0 + (NBATCH - 1) * NB, NB)],
            sem_o[1]).wait()

    return run(a_p, b_p, c_p, v_p, idx_flat)


def _out_body(rep_ref, wo_ref, bo_ref, o_ref):
    o_ref[...] = (jnp.dot(rep_ref[...], wo_ref[...],
                          preferred_element_type=_f32) + bo_ref[...])


def _out_mm(rep, W_out, b_out2):
    nblk = N // PB
    return pl.pallas_call(
        _out_body,
        grid=(nblk,),
        in_specs=[pl.BlockSpec((PB, D), lambda i: (i, 0)),
                  pl.BlockSpec((D, D), lambda i: (0, 0)),
                  pl.BlockSpec((1, D), lambda i: (0, 0))],
        out_specs=pl.BlockSpec((PB, D), lambda i: (i, 0)),
        out_shape=jax.ShapeDtypeStruct((N, D), _f32),
    )(rep, W_out, b_out2)


def kernel(x, pos, batch, W_lin, W_src, W_dst, W_pos, b_pos, W_out, b_out):
    del batch
    pos_p = jnp.pad(pos, ((0, 0), (0, D - 3)))
    post_p = jnp.pad(pos.T, ((0, D - 3), (0, CPAD - N)))
    W_pos_p = jnp.pad(W_pos, ((0, D - 3), (0, 0)))
    b_pos2 = b_pos[None, :]
    b_out2 = b_out[None, :]

    a, b, c, v = _precompute(x, pos_p, W_lin, W_src, W_dst, W_pos_p, b_pos2)
    idx = _knn(pos_p, post_p)

    pad = ((0, NPAD - N), (0, 0))
    a_p = jnp.pad(a, pad)
    b_p = jnp.pad(b, pad)
    c_p = jnp.pad(c, pad)
    v_p = jnp.pad(v, pad)
    idx_flat = jnp.pad(idx, pad).reshape(-1)

    rep = _gather_softmax(a_p, b_p, c_p, v_p, idx_flat)[:N]
    return _out_mm(rep, W_out, b_out2)
